# Initial kernel scaffold; baseline (speedup 1.0000x reference)
import jax
import jax.numpy as jnp
from jax.experimental import pallas as pl


def kernel(x, W_enc, b_enc):
    raise NotImplementedError("write your pallas kernel here")


# speedup vs baseline: 10.0177x; 10.0177x over previous
import jax
import jax.numpy as jnp
from jax.experimental import pallas as pl

K = 32
BR = 128


def _key_to_float(cand_u32):
    sign = jnp.uint32(0x80000000)
    u = jnp.where(cand_u32 >= sign, cand_u32 ^ sign, ~cand_u32)
    return jax.lax.bitcast_convert_type(u, jnp.float32)


def _topk_mask_kernel(x_ref, w_ref, b_ref, o_ref):
    z = jax.lax.dot_general(
        x_ref[...], w_ref[...],
        dimension_numbers=(((1,), (0,)), ((), ())),
        preferred_element_type=jnp.float32,
    ) + b_ref[...]
    o_ref[...] = z

    def step(i, t_key):
        bit = jax.lax.shift_left(jnp.uint32(1), jnp.uint32(31) - i.astype(jnp.uint32))
        cand = t_key | bit
        thresh = _key_to_float(cand)
        cnt = jnp.sum((o_ref[...] >= thresh).astype(jnp.float32), axis=1,
                      keepdims=True)
        return jnp.where(cnt >= K, cand, t_key)

    t_key = jax.lax.fori_loop(0, 32, step,
                              jnp.zeros((o_ref.shape[0], 1), jnp.uint32))
    thresh = _key_to_float(t_key)
    zz = o_ref[...]
    o_ref[...] = jnp.where(zz >= thresh, zz, 0.0)


def kernel(x, W_enc, b_enc):
    n_tok, d_in = x.shape
    d_dict = W_enc.shape[0]
    wt = W_enc.T
    b2 = b_enc.reshape(1, d_dict)
    return pl.pallas_call(
        _topk_mask_kernel,
        grid=(n_tok // BR,),
        in_specs=[
            pl.BlockSpec((BR, d_in), lambda i: (i, 0)),
            pl.BlockSpec((d_in, d_dict), lambda i: (0, 0)),
            pl.BlockSpec((1, d_dict), lambda i: (0, 0)),
        ],
        out_specs=pl.BlockSpec((BR, d_dict), lambda i: (i, 0)),
        out_shape=jax.ShapeDtypeStruct((n_tok, d_dict), jnp.float32),
    )(x, wt, b2)


# speedup vs baseline: 10.1888x; 1.0171x over previous
import jax
import jax.numpy as jnp
from jax.experimental import pallas as pl

K = 32
BR = 128


def _key_to_float(cand_u32):
    sign = jnp.uint32(0x80000000)
    u = jnp.where(cand_u32 >= sign, cand_u32 ^ sign, ~cand_u32)
    return jax.lax.bitcast_convert_type(u, jnp.float32)


def _topk_mask_kernel(x_ref, w_ref, b_ref, o_ref):
    z = jax.lax.dot_general(
        x_ref[...], w_ref[...],
        dimension_numbers=(((1,), (0,)), ((), ())),
        preferred_element_type=jnp.float32,
    ) + b_ref[...]
    o_ref[...] = z

    def step(i, t_key):
        bit = jax.lax.shift_left(jnp.uint32(1), jnp.uint32(31) - i.astype(jnp.uint32))
        cand = t_key | bit
        thresh = _key_to_float(cand)
        cnt = jnp.sum((o_ref[...] >= thresh).astype(jnp.float32), axis=1,
                      keepdims=True)
        return jnp.where(cnt >= K, cand, t_key)

    t_key = jax.lax.fori_loop(0, 32, step,
                              jnp.zeros((o_ref.shape[0], 1), jnp.uint32))
    thresh = _key_to_float(t_key)
    zz = o_ref[...]
    o_ref[...] = jnp.where(zz >= thresh, zz, 0.0)


def kernel(x, W_enc, b_enc):
    n_tok, d_in = x.shape
    d_dict = W_enc.shape[0]
    wt = W_enc.T.astype(jnp.bfloat16)
    xb = x.astype(jnp.bfloat16)
    b2 = b_enc.reshape(1, d_dict)
    return pl.pallas_call(
        _topk_mask_kernel,
        grid=(n_tok // BR,),
        in_specs=[
            pl.BlockSpec((BR, d_in), lambda i: (i, 0)),
            pl.BlockSpec((d_in, d_dict), lambda i: (0, 0)),
            pl.BlockSpec((1, d_dict), lambda i: (0, 0)),
        ],
        out_specs=pl.BlockSpec((BR, d_dict), lambda i: (i, 0)),
        out_shape=jax.ShapeDtypeStruct((n_tok, d_dict), jnp.float32),
    )(xb, wt, b2)
